# baseline (speedup 1.0000x reference)
import jax
import jax.numpy as jnp
from jax.experimental import pallas as pl
from jax.experimental.pallas import tpu as pltpu

B, S, D, OUT, E = 2, 4096, 768, 768, 8
TILE = 2048


def _moe_ffn_kernel(x_ref, wg_ref, bg_ref, wa_hbm, ba_ref, wb_hbm, bb_ref,
                    wc_hbm, bc_ref, o_ref, wa_buf, wb_buf, wc_buf, sem,
                    hg_ref):
    def start_ab(e, slot):
        pltpu.make_async_copy(wa_hbm.at[e], wa_buf.at[slot],
                              sem.at[0, slot]).start()
        pltpu.make_async_copy(wb_hbm.at[e], wb_buf.at[slot],
                              sem.at[1, slot]).start()

    def wait_ab(e, slot):
        pltpu.make_async_copy(wa_hbm.at[e], wa_buf.at[slot],
                              sem.at[0, slot]).wait()
        pltpu.make_async_copy(wb_hbm.at[e], wb_buf.at[slot],
                              sem.at[1, slot]).wait()

    def start_c(e, slot):
        pltpu.make_async_copy(wc_hbm.at[e], wc_buf.at[slot],
                              sem.at[2, slot]).start()

    def wait_c(e, slot):
        pltpu.make_async_copy(wc_hbm.at[e], wc_buf.at[slot],
                              sem.at[2, slot]).wait()

    pid = pl.program_id(0)
    shift = jax.lax.rem(6 * pid, 8)

    @pl.when(pid == 0)
    def _first_tile_prologue():
        start_ab(0, 0)
        start_ab(1, 1)
        start_c(0, 0)
        start_c(1, 1)

    logits = jnp.dot(x_ref[...], wg_ref[...],
                     preferred_element_type=jnp.float32)
    gates = jax.nn.softmax(logits + bg_ref[...], axis=-1)
    o_ref[...] = jnp.dot(gates, bc_ref[...],
                         preferred_element_type=jnp.float32)
    lane = jax.lax.broadcasted_iota(jnp.int32, gates.shape, 1)

    def gate_col(e):
        return jnp.sum(jnp.where(lane == e, gates, 0.0), axis=1,
                       keepdims=True)

    @pl.when(pid == 0)
    def _wait_ab0():
        wait_ab(shift, 0)

    def dots(e_p, slot):
        a = jnp.dot(x_ref[...], wa_buf[slot],
                    preferred_element_type=jnp.float32)
        a = a + ba_ref[e_p]
        b = jnp.dot(x_ref[...], wb_buf[slot],
                    preferred_element_type=jnp.float32)
        b = b + bb_ref[e_p]
        sig = 0.5 * (jnp.tanh(0.5 * a) + 1.0)
        hg_ref[...] = ((a * sig) * b * gate_col(e_p)).astype(jnp.bfloat16)

    dots(shift, 0)
    start_ab(jax.lax.rem(2 + shift, 8), 0)

    def body(p, carry):
        slot = jax.lax.rem(p, 2)
        e_p = jax.lax.rem(p + shift, 8)
        e_c = jax.lax.rem(p - 1 + shift, 8)

        @pl.when((pid == 0) | (p >= 2))
        def _wait_ab():
            wait_ab(e_p, slot)

        @pl.when((p >= 3) | (pid == 0))
        def _wait_c():
            wait_c(e_c, 1 - slot)

        o_ref[...] += jnp.dot(
            hg_ref[...], wc_buf[1 - slot].astype(jnp.bfloat16),
            preferred_element_type=jnp.float32)
        dots(e_p, slot)

        @pl.when(p <= 6)
        def _start_c():
            start_c(jax.lax.rem(p + 1 + shift, 8), 1 - slot)

        @pl.when(p <= 5)
        def _start_ab():
            start_ab(jax.lax.rem(p + 2 + shift, 8), slot)

        return carry

    jax.lax.fori_loop(1, E, body, 0, unroll=True)

    wait_c(jax.lax.rem(7 + shift, 8), 1)
    o_ref[...] += jnp.dot(hg_ref[...], wc_buf[1].astype(jnp.bfloat16),
                          preferred_element_type=jnp.float32)


@jax.jit
def _moe_ffn(x, Wg, bg, Wa, ba, Wb, bb, Wc, bc):
    n = x.shape[0]
    grid = (n // TILE,)
    return pl.pallas_call(
        _moe_ffn_kernel,
        grid=grid,
        in_specs=[
            pl.BlockSpec((TILE, D), lambda i: (i, 0)),
            pl.BlockSpec((D, E), lambda i: (0, 0)),
            pl.BlockSpec((1, E), lambda i: (0, 0)),
            pl.BlockSpec(memory_space=pl.ANY),
            pl.BlockSpec((E, 1, OUT), lambda i: (0, 0, 0)),
            pl.BlockSpec(memory_space=pl.ANY),
            pl.BlockSpec((E, 1, OUT), lambda i: (0, 0, 0)),
            pl.BlockSpec(memory_space=pl.ANY),
            pl.BlockSpec((E, OUT), lambda i: (0, 0)),
        ],
        out_specs=pl.BlockSpec((TILE, OUT), lambda i: (i, 0)),
        out_shape=jax.ShapeDtypeStruct((n, OUT), jnp.float32),
        compiler_params=pltpu.CompilerParams(
            vmem_limit_bytes=63 * 1024 * 1024),
        scratch_shapes=[
            pltpu.VMEM((2, D, OUT), jnp.float32),
            pltpu.VMEM((2, D, OUT), jnp.float32),
            pltpu.VMEM((2, OUT, OUT), jnp.float32),
            pltpu.SemaphoreType.DMA((3, 2)),
            pltpu.VMEM((TILE, OUT), jnp.bfloat16),
        ],
    )(x, Wg, bg, Wa, ba, Wb, bb, Wc, bc)


def kernel(inputs, Wg, bg, Wa, ba, Wb, bb, Wc, bc):
    b, s, d = inputs.shape
    x = inputs.reshape(b * s, d)
    out = _moe_ffn(x, Wg, bg.reshape(1, E), Wa, ba.reshape(E, 1, OUT), Wb,
                   bb.reshape(E, 1, OUT), Wc, bc)
    return out.reshape(b, s, OUT)


# speedup vs baseline: 1.3027x; 1.3027x over previous
import jax
import jax.numpy as jnp
from jax.experimental import pallas as pl
from jax.experimental.pallas import tpu as pltpu

B, S, D, OUT, E = 2, 4096, 768, 768, 8
TILE = 2048


def _moe_ffn_kernel(x_ref, wg_ref, bg_ref, wa_hbm, ba_ref, wb_hbm, bb_ref,
                    wc_hbm, bc_ref, o_ref, wa_buf, wb_buf, wc_buf, sem,
                    hg_ref):
    def start_ab(e, slot):
        pltpu.make_async_copy(wa_hbm.at[e], wa_buf.at[slot],
                              sem.at[0, slot]).start()
        pltpu.make_async_copy(wb_hbm.at[e], wb_buf.at[slot],
                              sem.at[1, slot]).start()

    def wait_ab(e, slot):
        pltpu.make_async_copy(wa_hbm.at[e], wa_buf.at[slot],
                              sem.at[0, slot]).wait()
        pltpu.make_async_copy(wb_hbm.at[e], wb_buf.at[slot],
                              sem.at[1, slot]).wait()

    def start_c(e, slot):
        pltpu.make_async_copy(wc_hbm.at[e], wc_buf.at[slot],
                              sem.at[2, slot]).start()

    def wait_c(e, slot):
        pltpu.make_async_copy(wc_hbm.at[e], wc_buf.at[slot],
                              sem.at[2, slot]).wait()

    pid = pl.program_id(0)
    shift = jax.lax.rem(6 * pid, 8)

    @pl.when(pid == 0)
    def _first_tile_prologue():
        start_ab(0, 0)
        start_ab(1, 1)
        start_c(0, 0)
        start_c(1, 1)

    logits = jnp.dot(x_ref[...], wg_ref[...],
                     preferred_element_type=jnp.float32)
    gates = jax.nn.softmax(logits + bg_ref[...], axis=-1)
    o_ref[...] = jnp.dot(gates, bc_ref[...],
                         preferred_element_type=jnp.float32)
    lane = jax.lax.broadcasted_iota(jnp.int32, gates.shape, 1)

    def gate_col(e):
        return jnp.sum(jnp.where(lane == e, gates, 0.0), axis=1,
                       keepdims=True)

    @pl.when(pid == 0)
    def _wait_ab0():
        wait_ab(shift, 0)

    def dots(e_p, slot):
        a = jnp.dot(x_ref[...], wa_buf[slot],
                    preferred_element_type=jnp.float32)
        a = a + ba_ref[e_p]
        b = jnp.dot(x_ref[...], wb_buf[slot],
                    preferred_element_type=jnp.float32)
        b = b + bb_ref[e_p]
        sig = 0.5 * (jnp.tanh(0.5 * a) + 1.0)
        hg_ref[...] = (a * sig) * b * gate_col(e_p)

    dots(shift, 0)
    start_ab(jax.lax.rem(2 + shift, 8), 0)

    def body(p, carry):
        slot = jax.lax.rem(p, 2)
        e_p = jax.lax.rem(p + shift, 8)
        e_c = jax.lax.rem(p - 1 + shift, 8)

        @pl.when((pid == 0) | (p >= 2))
        def _wait_ab():
            wait_ab(e_p, slot)

        @pl.when((p >= 3) | (pid == 0))
        def _wait_c():
            wait_c(e_c, 1 - slot)

        o_ref[...] += jnp.dot(hg_ref[...], wc_buf[1 - slot],
                              preferred_element_type=jnp.float32)
        dots(e_p, slot)

        @pl.when(p <= 6)
        def _start_c():
            start_c(jax.lax.rem(p + 1 + shift, 8), 1 - slot)

        @pl.when(p <= 5)
        def _start_ab():
            start_ab(jax.lax.rem(p + 2 + shift, 8), slot)

        return carry

    jax.lax.fori_loop(1, E, body, 0, unroll=False)

    wait_c(jax.lax.rem(7 + shift, 8), 1)
    o_ref[...] += jnp.dot(hg_ref[...], wc_buf[1],
                          preferred_element_type=jnp.float32)


@jax.jit
def _moe_ffn(x, Wg, bg, Wa, ba, Wb, bb, Wc, bc):
    n = x.shape[0]
    grid = (n // TILE,)
    return pl.pallas_call(
        _moe_ffn_kernel,
        grid=grid,
        in_specs=[
            pl.BlockSpec((TILE, D), lambda i: (i, 0)),
            pl.BlockSpec((D, E), lambda i: (0, 0)),
            pl.BlockSpec((1, E), lambda i: (0, 0)),
            pl.BlockSpec(memory_space=pl.ANY),
            pl.BlockSpec((E, 1, OUT), lambda i: (0, 0, 0)),
            pl.BlockSpec(memory_space=pl.ANY),
            pl.BlockSpec((E, 1, OUT), lambda i: (0, 0, 0)),
            pl.BlockSpec(memory_space=pl.ANY),
            pl.BlockSpec((E, OUT), lambda i: (0, 0)),
        ],
        out_specs=pl.BlockSpec((TILE, OUT), lambda i: (i, 0)),
        out_shape=jax.ShapeDtypeStruct((n, OUT), jnp.float32),
        compiler_params=pltpu.CompilerParams(
            vmem_limit_bytes=63 * 1024 * 1024),
        scratch_shapes=[
            pltpu.VMEM((2, D, OUT), jnp.float32),
            pltpu.VMEM((2, D, OUT), jnp.float32),
            pltpu.VMEM((2, OUT, OUT), jnp.float32),
            pltpu.SemaphoreType.DMA((3, 2)),
            pltpu.VMEM((TILE, OUT), jnp.float32),
        ],
    )(x, Wg, bg, Wa, ba, Wb, bb, Wc, bc)


def kernel(inputs, Wg, bg, Wa, ba, Wb, bb, Wc, bc):
    b, s, d = inputs.shape
    x = inputs.reshape(b * s, d)
    out = _moe_ffn(x, Wg, bg.reshape(1, E), Wa, ba.reshape(E, 1, OUT), Wb,
                   bb.reshape(E, 1, OUT), Wc, bc)
    return out.reshape(b, s, OUT)
